# baseline (speedup 1.0000x reference)
import jax
import jax.numpy as jnp
from jax import lax
from jax.experimental import pallas as pl
from jax.experimental.pallas import tpu as pltpu
from jax.experimental.pallas import tpu_sc as plsc

NUM_TOTEMS = 1000000
POS_DIM = 16
BATCH = 16384

_NC = 2
_NS = 16
_NW = _NC * _NS
_BPW = BATCH // _NW
_L = 16


def _tile_body(idx_hbm, tblA, tblB, outT, idx_v, o_a, o_b, sems):
    wid = lax.axis_index("s") * _NC + lax.axis_index("c")
    base = wid * _BPW
    pltpu.sync_copy(idx_hbm.at[pl.ds(base, _BPW)], idx_v)

    copies = []
    for dst, tbl in ((o_a, tblA), (o_b, tblB)):
        for c in range(POS_DIM):
            row = tbl.at[c // 8, c % 8]
            copies.append(
                pltpu.async_copy(row.at[idx_v], dst.at[c], sems.at[c % 2]))
    for cp in copies:
        cp.wait()

    def add_rows(k, carry):
        d = pl.ds(k * _L, _L)
        for c in range(POS_DIM):
            o_a[c, d] = o_a[c, d] + o_b[c, d]
        return carry

    lax.fori_loop(0, _BPW // _L, add_rows, 0, unroll=4)
    pltpu.sync_copy(o_a, outT.at[:, pl.ds(base, _BPW)])


@jax.jit
def _lookup(totem_id, tblA, tblB):
    mesh = plsc.VectorSubcoreMesh(core_axis_name="c", subcore_axis_name="s")
    outT = pl.kernel(
        _tile_body,
        mesh=mesh,
        compiler_params=pltpu.CompilerParams(use_tc_tiling_on_sc=False),
        out_type=jax.ShapeDtypeStruct((POS_DIM, BATCH), jnp.float32),
        scratch_types=[
            pltpu.VMEM((_BPW,), jnp.int32),
            pltpu.VMEM((POS_DIM, _BPW), jnp.float32),
            pltpu.VMEM((POS_DIM, _BPW), jnp.float32),
            pltpu.SemaphoreType.DMA((2,)),
        ],
    )(totem_id, tblA, tblB)
    return outT.T


def kernel(totem_id, init_totem_pos, totem_pos_residual):
    tblA = init_totem_pos.T.reshape(2, 8, NUM_TOTEMS)
    tblB = totem_pos_residual.T.reshape(2, 8, NUM_TOTEMS)
    return _lookup(totem_id.astype(jnp.int32), tblA, tblB)


# speedup vs baseline: 2.8806x; 2.8806x over previous
import jax
import jax.numpy as jnp
from jax import lax
from jax.experimental import pallas as pl
from jax.experimental.pallas import tpu as pltpu
from jax.experimental.pallas import tpu_sc as plsc

NUM_TOTEMS = 1000000
POS_DIM = 16
BATCH = 16384

_NC = 2
_NS = 16
_NW = _NC * _NS
_BPW = BATCH // _NW
_L = 16
_CHUNK = 256
_NCHUNK = _BPW // _CHUNK


def _tile_body(idx_hbm, tbl, outT, idx_v, row0, row1, rA, rB, o_v,
               sems):
    tblA = tbl.at[0]
    tblB = tbl.at[1]
    wid = lax.axis_index("s") * _NC + lax.axis_index("c")
    base = wid * _BPW
    pltpu.sync_copy(idx_hbm.at[pl.ds(base, _BPW)], idx_v)

    row_bufs = (row0, row1)

    def fill_rows(k, carry):
        for chunk in range(_NCHUNK):
            r = idx_v[pl.ds(chunk * _CHUNK + k * _L, _L)]
            row_bufs[chunk][pl.ds(k * _L, _L)] = r >> 3
        return carry

    lax.fori_loop(0, _CHUNK // _L, fill_rows, 0, unroll=4)

    lanes = lax.iota(jnp.int32, _L)
    for chunk in range(_NCHUNK):
        cpA = pltpu.async_copy(tblA.at[row_bufs[chunk]], rA, sems.at[0])
        cpB = pltpu.async_copy(tblB.at[row_bufs[chunk]], rB, sems.at[1])
        cpA.wait()
        cpB.wait()

        def extract(g, carry):
            ids = idx_v[pl.ds(chunk * _CHUNK + g * _L, _L)]
            lane0 = (ids & 7) * 16
            rows = lanes + g * _L
            for c in range(POS_DIM):
                vals = plsc.load_gather(rA, [rows, lane0 + c])
                vals = vals + plsc.load_gather(rB, [rows, lane0 + c])
                o_v[c, pl.ds(chunk * _CHUNK + g * _L, _L)] = vals
            return carry

        lax.fori_loop(0, _CHUNK // _L, extract, 0, unroll=2)

    pltpu.sync_copy(o_v, outT.at[:, pl.ds(base, _BPW)])


@jax.jit
def _lookup(totem_id, tbl):
    mesh = plsc.VectorSubcoreMesh(core_axis_name="c", subcore_axis_name="s")
    outT = pl.kernel(
        _tile_body,
        mesh=mesh,
        compiler_params=pltpu.CompilerParams(needs_layout_passes=False),
        out_type=jax.ShapeDtypeStruct((POS_DIM, BATCH), jnp.float32),
        scratch_types=[
            pltpu.VMEM((_BPW,), jnp.int32),
            pltpu.VMEM((_CHUNK,), jnp.int32),
            pltpu.VMEM((_CHUNK,), jnp.int32),
            pltpu.VMEM((_CHUNK, 128), jnp.float32),
            pltpu.VMEM((_CHUNK, 128), jnp.float32),
            pltpu.VMEM((POS_DIM, _BPW), jnp.float32),
            pltpu.SemaphoreType.DMA((2,)),
        ],
    )(totem_id, tbl)
    return outT.T


def kernel(totem_id, init_totem_pos, totem_pos_residual):
    tbl = jnp.stack([
        init_totem_pos.reshape(NUM_TOTEMS * POS_DIM // 128, 128),
        totem_pos_residual.reshape(NUM_TOTEMS * POS_DIM // 128, 128),
    ])
    return _lookup(totem_id.astype(jnp.int32), tbl)


# speedup vs baseline: 3.1588x; 1.0966x over previous
import jax
import jax.numpy as jnp
from jax import lax
from jax.experimental import pallas as pl
from jax.experimental.pallas import tpu as pltpu
from jax.experimental.pallas import tpu_sc as plsc

NUM_TOTEMS = 1000000
POS_DIM = 16
BATCH = 16384

_NC = 2
_NS = 16
_NW = _NC * _NS
_BPW = BATCH // _NW
_L = 16
_CHUNK = 256
_NCHUNK = _BPW // _CHUNK


def _tile_body(idx_hbm, tblA, tblB, outT, idx_v, row0, row1, rA, rB, o_v,
               sems):
    wid = lax.axis_index("s") * _NC + lax.axis_index("c")
    base = wid * _BPW
    pltpu.sync_copy(idx_hbm.at[pl.ds(base, _BPW)], idx_v)

    row_bufs = (row0, row1)

    def fill_rows(k, carry):
        for chunk in range(_NCHUNK):
            r = idx_v[pl.ds(chunk * _CHUNK + k * _L, _L)]
            row_bufs[chunk][pl.ds(k * _L, _L)] = r >> 3
        return carry

    lax.fori_loop(0, _CHUNK // _L, fill_rows, 0, unroll=4)

    lanes = lax.iota(jnp.int32, _L)
    for chunk in range(_NCHUNK):
        cpA = pltpu.async_copy(tblA.at[row_bufs[chunk]], rA, sems.at[0])
        cpB = pltpu.async_copy(tblB.at[row_bufs[chunk]], rB, sems.at[1])
        cpA.wait()
        cpB.wait()

        def extract(g, carry):
            ids = idx_v[pl.ds(chunk * _CHUNK + g * _L, _L)]
            lane0 = (ids & 7) * 16
            rows = lanes + g * _L
            for c in range(POS_DIM):
                vals = plsc.load_gather(rA, [rows, lane0 + c])
                vals = vals + plsc.load_gather(rB, [rows, lane0 + c])
                o_v[c, pl.ds(chunk * _CHUNK + g * _L, _L)] = vals
            return carry

        lax.fori_loop(0, _CHUNK // _L, extract, 0, unroll=2)

    pltpu.sync_copy(o_v, outT.at[:, pl.ds(base, _BPW)])


@jax.jit
def _lookup(totem_id, tblA, tblB):
    mesh = plsc.VectorSubcoreMesh(core_axis_name="c", subcore_axis_name="s")
    outT = pl.kernel(
        _tile_body,
        mesh=mesh,
        compiler_params=pltpu.CompilerParams(needs_layout_passes=False),
        out_type=jax.ShapeDtypeStruct((POS_DIM, BATCH), jnp.float32),
        scratch_types=[
            pltpu.VMEM((_BPW,), jnp.int32),
            pltpu.VMEM((_CHUNK,), jnp.int32),
            pltpu.VMEM((_CHUNK,), jnp.int32),
            pltpu.VMEM((_CHUNK, 128), jnp.float32),
            pltpu.VMEM((_CHUNK, 128), jnp.float32),
            pltpu.VMEM((POS_DIM, _BPW), jnp.float32),
            pltpu.SemaphoreType.DMA((2,)),
        ],
    )(totem_id, tblA, tblB)
    return outT.T


def kernel(totem_id, init_totem_pos, totem_pos_residual):
    tblA = init_totem_pos.reshape(NUM_TOTEMS * POS_DIM // 128, 128)
    tblB = totem_pos_residual.reshape(NUM_TOTEMS * POS_DIM // 128, 128)
    return _lookup(totem_id.astype(jnp.int32), tblA, tblB)
